# Initial kernel scaffold; baseline (speedup 1.0000x reference)
import jax
import jax.numpy as jnp
from jax.experimental import pallas as pl


def kernel(input, index):
    raise NotImplementedError("write your pallas kernel here")


# speedup vs baseline: 3.8417x; 3.8417x over previous
import functools

import jax
import jax.numpy as jnp
from jax import lax
from jax.experimental import pallas as pl
from jax.experimental.pallas import tpu as pltpu
from jax.experimental.pallas import tpu_sc as plsc

_N = 320000
_F = 128
_S = 10000
_ST = 10240
_EPS = 1e-9

_NW = 32
_RPW = _N // _NW
_G = 80
_GPW = _RPW // _G
_GPS = 5
_NSTEP = _GPW // _GPS

_mesh = plsc.VectorSubcoreMesh(core_axis_name="c", subcore_axis_name="s")


def _phase_a_body(x_hbm, idx_hbm, z128_hbm, ones_hbm,
                  psum_hbm, pcnt_hbm,
                  idx_v, bufs, lsem, asem, sums_sh):
  cid = lax.axis_index("c")
  sid = lax.axis_index("s")
  wid = sid * 2 + cid

  @pl.when(sid == 0)
  def _zero_table():
    pltpu.sync_copy(z128_hbm, sums_sh)

  plsc.subcore_barrier()

  row0 = wid * _RPW

  def _x_slice(grp):
    base = pl.multiple_of(row0 + grp * _G, 8)
    return x_hbm.at[pl.ds(base, _G)]

  pltpu.async_copy(_x_slice(0), bufs.at[0], lsem)

  @pl.loop(0, _NSTEP)
  def _scatter_loop(step):
    pltpu.sync_copy(idx_hbm.at[wid, step], idx_v)
    for g in range(_GPS):
      grp = step * _GPS + g
      par = (step + g) % 2
      nxt = jnp.minimum(grp + 1, _GPW - 1)
      pltpu.async_copy(_x_slice(nxt), bufs.at[1 - par], lsem)
      pltpu.make_async_copy(_x_slice(grp), bufs.at[par], lsem).wait()
      pltpu.sync_copy(bufs.at[par], sums_sh.at[idx_v.at[g]], add=True)

  pltpu.make_async_copy(_x_slice(0), bufs.at[0], lsem).wait()
  plsc.subcore_barrier()

  @pl.when(sid == 0)
  def _write_sums():
    pltpu.sync_copy(sums_sh, psum_hbm.at[cid])
    pltpu.sync_copy(z128_hbm, sums_sh)

  pltpu.sync_copy(ones_hbm, bufs.at[0])
  plsc.subcore_barrier()

  @pl.loop(0, _NSTEP)
  def _count_loop(step):
    pltpu.sync_copy(idx_hbm.at[wid, step], idx_v)
    descs = [
        pltpu.async_copy(bufs.at[0], sums_sh.at[idx_v.at[g]], asem, add=True)
        for g in range(_GPS)
    ]
    for d in descs:
      d.wait()

  plsc.subcore_barrier()

  @pl.when(sid == 0)
  def _write_counts():
    pltpu.sync_copy(sums_sh, pcnt_hbm.at[cid])


_phase_a = functools.partial(
    pl.kernel,
    out_type=[
        jax.ShapeDtypeStruct((2, _ST, _F), jnp.float32),
        jax.ShapeDtypeStruct((2, _ST, _F), jnp.float32),
    ],
    mesh=_mesh,
    scratch_types=[
        pltpu.VMEM((_GPS, _G), jnp.int32),
        pltpu.VMEM((2, _G, _F), jnp.float32),
        pltpu.SemaphoreType.DMA,
        pltpu.SemaphoreType.DMA,
        pltpu.VMEM_SHARED((_ST, _F), jnp.float32),
    ],
)(_phase_a_body)


def _combine_body(p0_ref, p1_ref, c0_ref, c1_ref, out_ref):
  cnt = c0_ref[:, 0:1] + c1_ref[:, 0:1]
  out_ref[...] = (p0_ref[...] + p1_ref[...]) / (cnt + _EPS)


def _phase_c_body(pooled_hbm, idx_hbm, out_hbm, idx_v, bufs, gsem, ssem):
  cid = lax.axis_index("c")
  sid = lax.axis_index("s")
  wid = sid * 2 + cid
  row0 = wid * _RPW

  def _out_slice(grp):
    base = pl.multiple_of(row0 + grp * _G, 8)
    return out_hbm.at[pl.ds(base, _G)]

  @pl.loop(0, _NSTEP)
  def _gather_loop(step):
    pltpu.sync_copy(idx_hbm.at[wid, step], idx_v)
    bank = (step % 2) * _GPS

    @pl.when(step >= 2)
    def _drain_old_stores():
      for _ in range(_GPS):
        pltpu.make_async_copy(bufs.at[0], _out_slice(0), ssem).wait()

    descs = [
        pltpu.async_copy(pooled_hbm.at[idx_v.at[g]], bufs.at[bank + g], gsem)
        for g in range(_GPS)
    ]
    for d in descs:
      d.wait()
    for g in range(_GPS):
      pltpu.async_copy(bufs.at[bank + g], _out_slice(step * _GPS + g), ssem)

  for _ in range(2 * _GPS):
    pltpu.make_async_copy(bufs.at[0], _out_slice(0), ssem).wait()


_phase_c = functools.partial(
    pl.kernel,
    out_type=jax.ShapeDtypeStruct((_N, _F), jnp.float32),
    mesh=_mesh,
    scratch_types=[
        pltpu.VMEM((_GPS, _G), jnp.int32),
        pltpu.VMEM((2 * _GPS, _G, _F), jnp.float32),
        pltpu.SemaphoreType.DMA,
        pltpu.SemaphoreType.DMA,
    ],
)(_phase_c_body)


@jax.jit
def kernel(input, index):
  idx4d = index.astype(jnp.int32).reshape(_NW, _NSTEP, _GPS, _G)
  z128 = jnp.zeros((_ST, _F), jnp.float32)
  ones128 = jnp.ones((_G, _F), jnp.float32)

  psum, pcnt = _phase_a(input, idx4d, z128, ones128)

  pooled = pl.pallas_call(
      _combine_body,
      out_shape=jax.ShapeDtypeStruct((_ST, _F), jnp.float32),
  )(psum[0], psum[1], pcnt[0], pcnt[1])

  return _phase_c(pooled, idx4d)
